# Initial kernel scaffold; baseline (speedup 1.0000x reference)
import jax
import jax.numpy as jnp
from jax.experimental import pallas as pl


def kernel(x, target_num_token):
    raise NotImplementedError("write your pallas kernel here")


# speedup vs baseline: 1.0054x; 1.0054x over previous
import functools

import jax
import jax.numpy as jnp
from jax import lax
from jax.experimental import pallas as pl

_B, _P, _C = 8, 2048, 1024
_H = 16
_D = _C // _H
_T1 = _P // 2
_T2 = _T1 // 2
_R2 = 295
_U2 = _T2 - _R2

_F32 = jnp.float32
_HI = lax.Precision.HIGHEST


def _rowmin_idx(vals, axis):
    mx = jnp.max(vals, axis=axis, keepdims=True)
    iota = lax.broadcasted_iota(jnp.int32, vals.shape, axis)
    return jnp.min(jnp.where(vals == mx, iota, 2 ** 30), axis=axis,
                   keepdims=True)


def _metric(x3d):
    m = jnp.mean(x3d, axis=1)
    n = jnp.sqrt(jnp.sum(m * m, axis=-1, keepdims=True))
    return m / n


def _kernel_a(xe_ref, xo_ref, nidx_ref):
    mn_e = _metric(xe_ref[0])
    mn_o = _metric(xo_ref[0])
    scores_t = lax.dot_general(mn_o, mn_e, (((1,), (1,)), ((), ())),
                               preferred_element_type=_F32)
    nidx_ref[0] = _rowmin_idx(scores_t, 0)


def _kernel_b(xe_ref, xo_ref, nidx_ref, x1w_ref, sz1_ref):
    idx_row = nidx_ref[0]
    iota_s = lax.broadcasted_iota(jnp.int32, (_T1, _T1), 0)
    m1t = (idx_row == iota_s).astype(_F32)
    x1w_ref[0] = xo_ref[0] + lax.dot_general(
        m1t, xe_ref[0], (((1,), (0,)), ((), ())),
        precision=_HI, preferred_element_type=_F32)
    ones = jnp.ones((_T1, 1), _F32)
    sz1_ref[0] = 1.0 + lax.dot_general(
        m1t, ones, (((1,), (0,)), ((), ())),
        precision=_HI, preferred_element_type=_F32)


def _kernel_c(xer_ref, xor_ref, xe_ref, xo_ref, sze_ref, szo_ref,
              unm_ref, dst_ref):
    sze = sze_ref[0]
    szo = szo_ref[0]
    mn_e = _metric(xer_ref[0] / sze[:, :, None])
    mn_o = _metric(xor_ref[0] / szo[:, :, None])
    scores = lax.dot_general(mn_e, mn_o, (((1,), (1,)), ((), ())),
                             preferred_element_type=_F32)
    scores_t = lax.dot_general(mn_o, mn_e, (((1,), (1,)), ((), ())),
                               preferred_element_type=_F32)
    nmax_col = jnp.max(scores, axis=1, keepdims=True)
    nmax_row = jnp.max(scores_t, axis=0, keepdims=True)
    nidx_row = _rowmin_idx(scores_t, 0)
    gt = (nmax_col > nmax_row).astype(jnp.int32)
    iota_j = lax.broadcasted_iota(jnp.int32, (_T2, _T2), 0)
    iota_t = lax.broadcasted_iota(jnp.int32, (_T2, _T2), 1)
    eqlt = ((nmax_col == nmax_row) & (iota_j < iota_t)).astype(jnp.int32)
    rank_row = jnp.sum(gt + eqlt, axis=0, keepdims=True)
    merged_row = rank_row < _R2

    x2e = (xe_ref[0] / sze) * sze
    x2o = (xo_ref[0] / szo) * szo

    iota_s2 = lax.broadcasted_iota(jnp.int32, (_T2, _T2), 0)
    m2t = (merged_row & (nidx_row == iota_s2)).astype(_F32)
    num_dst = x2o + lax.dot_general(m2t, x2e, (((1,), (0,)), ((), ())),
                                    precision=_HI, preferred_element_type=_F32)
    den_dst = szo + lax.dot_general(m2t, sze, (((1,), (0,)), ((), ())),
                                    precision=_HI, preferred_element_type=_F32)
    dst_ref[0] = num_dst / den_dst

    iota_u = lax.broadcasted_iota(jnp.int32, (_U2, _T2), 0)
    ut = ((~merged_row) & (rank_row - _R2 == iota_u)).astype(_F32)
    num_unm = lax.dot_general(ut, x2e, (((1,), (0,)), ((), ())),
                              precision=_HI, preferred_element_type=_F32)
    den_unm = lax.dot_general(ut, sze, (((1,), (0,)), ((), ())),
                              precision=_HI, preferred_element_type=_F32)
    unm_ref[0] = num_unm / den_unm


def kernel(x, target_num_token):
    del target_num_token
    xe = x[:, ::2, :]
    xo = x[:, 1::2, :]
    xer = xe.reshape(_B, _T1, _H, _D)
    xor = xo.reshape(_B, _T1, _H, _D)

    spec3 = lambda s1, s2: pl.BlockSpec((1, s1, s2), lambda i: (i, 0, 0))
    spec4 = pl.BlockSpec((1, _T1, _H, _D), lambda i: (i, 0, 0, 0))

    nidx1 = pl.pallas_call(
        _kernel_a,
        grid=(_B,),
        in_specs=[spec4, spec4],
        out_specs=spec3(1, _T1),
        out_shape=jax.ShapeDtypeStruct((_B, 1, _T1), jnp.int32),
    )(xer, xor)

    x1w, sz1 = pl.pallas_call(
        _kernel_b,
        grid=(_B,),
        in_specs=[spec3(_T1, _C), spec3(_T1, _C), spec3(1, _T1)],
        out_specs=[spec3(_T1, _C), spec3(_T1, 1)],
        out_shape=[jax.ShapeDtypeStruct((_B, _T1, _C), _F32),
                   jax.ShapeDtypeStruct((_B, _T1, 1), _F32)],
    )(xe, xo, nidx1)

    x1w_e = x1w[:, ::2, :]
    x1w_o = x1w[:, 1::2, :]
    x1w_er = x1w_e.reshape(_B, _T2, _H, _D)
    x1w_or = x1w_o.reshape(_B, _T2, _H, _D)
    sz_e = sz1[:, ::2, :]
    sz_o = sz1[:, 1::2, :]

    spec4b = pl.BlockSpec((1, _T2, _H, _D), lambda i: (i, 0, 0, 0))
    out_unm, out_dst = pl.pallas_call(
        _kernel_c,
        grid=(_B,),
        in_specs=[spec4b, spec4b, spec3(_T2, _C), spec3(_T2, _C),
                  spec3(_T2, 1), spec3(_T2, 1)],
        out_specs=[spec3(_U2, _C), spec3(_T2, _C)],
        out_shape=[jax.ShapeDtypeStruct((_B, _U2, _C), _F32),
                   jax.ShapeDtypeStruct((_B, _T2, _C), _F32)],
    )(x1w_er, x1w_or, x1w_e, x1w_o, sz_e, sz_o)

    return jnp.concatenate([out_unm, out_dst], axis=1)


# speedup vs baseline: 1.3285x; 1.3214x over previous
import jax
import jax.numpy as jnp
from jax import lax
from jax.experimental import pallas as pl

_B, _P, _C = 8, 2048, 1024
_H = 16
_D = _C // _H
_T1 = _P // 2
_T2 = _T1 // 2
_R2 = 295
_U2 = _T2 - _R2

_F32 = jnp.float32
_HI = lax.Precision.HIGHEST


def _rowmin_idx0(vals):
    mx = jnp.max(vals, axis=0, keepdims=True)
    iota = lax.broadcasted_iota(jnp.int32, vals.shape, 0)
    return jnp.min(jnp.where(vals == mx, iota, 2 ** 30), axis=0,
                   keepdims=True)


def _metric(x3d):
    m = jnp.mean(x3d, axis=1)
    n = jnp.sqrt(jnp.sum(m * m, axis=-1, keepdims=True))
    return m / n


def _dot(a, b, prec=None):
    return lax.dot_general(a, b, (((1,), (0,)), ((), ())), precision=prec,
                           preferred_element_type=_F32)


def _dot_t(a, b):
    return lax.dot_general(a, b, (((1,), (1,)), ((), ())),
                           preferred_element_type=_F32)


def _decide1(xe4_ref, xo4_ref, nidx_ref, sze_ref, szo_ref):
    m_e = _metric(xe4_ref[0, :, 0])
    m_o = _metric(xo4_ref[0, :, 0])
    scores_t = _dot_t(m_o, m_e)
    nidx_row = _rowmin_idx0(scores_t)
    nidx_ref[0] = nidx_row
    iota_q = lax.broadcasted_iota(jnp.int32, (_T2, _T1), 0)
    m1t_e = (nidx_row == 2 * iota_q).astype(_F32)
    m1t_o = (nidx_row == 2 * iota_q + 1).astype(_F32)
    ones1 = jnp.ones((_T1, 1), _F32)
    sze_ref[0] = 1.0 + _dot(m1t_e, ones1, _HI)
    szo_ref[0] = 1.0 + _dot(m1t_o, ones1, _HI)


_CK = 2
_HK = _H // _CK
_CC = _C // _CK


def _merge1(x4_ref, nidx_ref, x1we_ref, x1wo_ref):
    xc = x4_ref[0].reshape(_P, _CC)
    xe = xc.reshape(_T1, 2, _CC)[:, 0, :]
    rs4 = xc.reshape(_T2, 4, _CC)
    xo_e = rs4[:, 1, :]
    xo_o = rs4[:, 3, :]
    nidx_row = nidx_ref[0]
    iota_q = lax.broadcasted_iota(jnp.int32, (_T2, _T1), 0)
    m1t_e = (nidx_row == 2 * iota_q).astype(_F32)
    m1t_o = (nidx_row == 2 * iota_q + 1).astype(_F32)
    x1we_ref[0] = xo_e + _dot(m1t_e, xe, _HI)
    x1wo_ref[0] = xo_o + _dot(m1t_o, xe, _HI)


def _round2(x1we4_ref, x1wo4_ref, x1we_ref, x1wo_ref, sze_ref, szo_ref,
            unm_ref, dst_ref):
    sze = sze_ref[0]
    szo = szo_ref[0]
    x1e = x1we_ref[0] / sze
    x1o = x1wo_ref[0] / szo
    mn_e = _metric(x1we4_ref[0] / sze[:, :, None])
    mn_o = _metric(x1wo4_ref[0] / szo[:, :, None])
    scores = _dot_t(mn_e, mn_o)
    scores_t2 = _dot_t(mn_o, mn_e)
    nmax_col = jnp.max(scores, axis=1, keepdims=True)
    nmax_row = jnp.max(scores_t2, axis=0, keepdims=True)
    nidx_row2 = _rowmin_idx0(scores_t2)
    gt = (nmax_col > nmax_row).astype(jnp.int32)
    iota_j = lax.broadcasted_iota(jnp.int32, (_T2, _T2), 0)
    iota_t = lax.broadcasted_iota(jnp.int32, (_T2, _T2), 1)
    eqlt = ((nmax_col == nmax_row) & (iota_j < iota_t)).astype(jnp.int32)
    rank_row = jnp.sum(gt + eqlt, axis=0, keepdims=True)
    merged_row = rank_row < _R2

    x2e = x1e * sze
    x2o = x1o * szo

    iota_s2 = lax.broadcasted_iota(jnp.int32, (_T2, _T2), 0)
    m2t = (merged_row & (nidx_row2 == iota_s2)).astype(_F32)
    num_dst = x2o + _dot(m2t, x2e, _HI)
    den_dst = szo + _dot(m2t, sze, _HI)
    dst_ref[0] = num_dst / den_dst

    iota_u = lax.broadcasted_iota(jnp.int32, (_U2, _T2), 0)
    ut = ((~merged_row) & (rank_row - _R2 == iota_u)).astype(_F32)
    num_unm = _dot(ut, x2e, _HI)
    den_unm = _dot(ut, sze, _HI)
    unm_ref[0] = num_unm / den_unm


def kernel(x, target_num_token):
    del target_num_token
    x4 = x.reshape(_B, _P, _H, _D)

    x5 = x.reshape(_B, _T1, 2, _H, _D)

    spec = lambda s1, s2: pl.BlockSpec((1, s1, s2), lambda i: (i, 0, 0))
    nidx, sze, szo = pl.pallas_call(
        _decide1,
        grid=(_B,),
        in_specs=[pl.BlockSpec((1, _T1, 1, _H, _D), lambda i: (i, 0, 0, 0, 0)),
                  pl.BlockSpec((1, _T1, 1, _H, _D), lambda i: (i, 0, 1, 0, 0))],
        out_specs=[spec(1, _T1), spec(_T2, 1), spec(_T2, 1)],
        out_shape=[jax.ShapeDtypeStruct((_B, 1, _T1), jnp.int32),
                   jax.ShapeDtypeStruct((_B, _T2, 1), _F32),
                   jax.ShapeDtypeStruct((_B, _T2, 1), _F32)],
    )(x5, x5)

    x1we, x1wo = pl.pallas_call(
        _merge1,
        grid=(_B, _CK),
        in_specs=[pl.BlockSpec((1, _P, _HK, _D), lambda i, j: (i, 0, j, 0)),
                  pl.BlockSpec((1, 1, _T1), lambda i, j: (i, 0, 0))],
        out_specs=[pl.BlockSpec((1, _T2, _CC), lambda i, j: (i, 0, j)),
                   pl.BlockSpec((1, _T2, _CC), lambda i, j: (i, 0, j))],
        out_shape=[jax.ShapeDtypeStruct((_B, _T2, _C), _F32),
                   jax.ShapeDtypeStruct((_B, _T2, _C), _F32)],
    )(x4, nidx)

    spec4 = pl.BlockSpec((1, _T2, _H, _D), lambda i: (i, 0, 0, 0))
    out_unm, out_dst = pl.pallas_call(
        _round2,
        grid=(_B,),
        in_specs=[spec4, spec4,
                  spec(_T2, _C), spec(_T2, _C), spec(_T2, 1), spec(_T2, 1)],
        out_specs=[spec(_U2, _C), spec(_T2, _C)],
        out_shape=[jax.ShapeDtypeStruct((_B, _U2, _C), _F32),
                   jax.ShapeDtypeStruct((_B, _T2, _C), _F32)],
    )(x1we.reshape(_B, _T2, _H, _D), x1wo.reshape(_B, _T2, _H, _D),
      x1we, x1wo, sze, szo)

    return jnp.concatenate([out_unm, out_dst], axis=1)


# speedup vs baseline: 1.6578x; 1.2479x over previous
import jax
import jax.numpy as jnp
from jax import lax
from jax.experimental import pallas as pl
from jax.experimental.pallas import tpu as pltpu

_B, _P, _C = 8, 2048, 1024
_H = 16
_D = _C // _H
_T1 = _P // 2
_T2 = _T1 // 2
_R2 = 295
_U2 = _T2 - _R2

_F32 = jnp.float32
_HI = lax.Precision.HIGHEST


def _rowmin_idx0(vals):
    mx = jnp.max(vals, axis=0, keepdims=True)
    iota = lax.broadcasted_iota(jnp.int32, vals.shape, 0)
    return jnp.min(jnp.where(vals == mx, iota, 2 ** 30), axis=0,
                   keepdims=True)


def _rowmin_idx1(vals):
    mx = jnp.max(vals, axis=1, keepdims=True)
    iota = lax.broadcasted_iota(jnp.int32, vals.shape, 1)
    return jnp.min(jnp.where(vals == mx, iota, 2 ** 30), axis=1,
                   keepdims=True)


def _metric(x3d):
    m = jnp.mean(x3d, axis=1)
    n = jnp.sqrt(jnp.sum(m * m, axis=-1, keepdims=True))
    return m / n


def _dot(a, b, prec=None):
    return lax.dot_general(a, b, (((1,), (0,)), ((), ())), precision=prec,
                           preferred_element_type=_F32)


def _dot_t(a, b, prec=None):
    return lax.dot_general(a, b, (((1,), (1,)), ((), ())), precision=prec,
                           preferred_element_type=_F32)


def _row_of_col(col):
    n = col.shape[0]
    eye = (lax.broadcasted_iota(jnp.int32, (n, n), 0) ==
           lax.broadcasted_iota(jnp.int32, (n, n), 1)).astype(_F32)
    return lax.dot_general(col, eye, (((0,), (0,)), ((), ())),
                           precision=_HI, preferred_element_type=_F32)


def _decide1(xe4_ref, xo4_ref, perm_ref, dstp_ref, sze_ref, szo_ref):
    m_e = _metric(xe4_ref[0, :, 0])
    m_o = _metric(xo4_ref[0, :, 0])
    scores = _dot_t(m_e, m_o)
    scores_t = _dot_t(m_o, m_e)
    nidx_row = _rowmin_idx0(scores_t)
    nmax_col = jnp.max(scores, axis=1, keepdims=True)
    nmax_row = _row_of_col(nmax_col)
    iota_j = lax.broadcasted_iota(jnp.int32, (_T1, _T1), 0)
    iota_t = lax.broadcasted_iota(jnp.int32, (_T1, _T1), 1)
    gt = (nmax_col > nmax_row).astype(jnp.int32)
    eqlt = ((nmax_col == nmax_row) & (iota_j < iota_t)).astype(jnp.int32)
    rank_row = jnp.sum(gt + eqlt, axis=0, keepdims=True)
    pmat = (rank_row == iota_j).astype(_F32)
    tval = lax.broadcasted_iota(jnp.int32, (1, _T1), 1).astype(_F32)
    perm_ref[0] = _dot_t(tval, pmat, _HI).astype(jnp.int32)
    dstp_ref[0] = _dot_t(nidx_row.astype(_F32), pmat,
                         _HI).astype(jnp.int32)
    iota_q = lax.broadcasted_iota(jnp.int32, (_T2, _T1), 0)
    m1t_e = (nidx_row == 2 * iota_q).astype(_F32)
    m1t_o = (nidx_row == 2 * iota_q + 1).astype(_F32)
    ones1 = jnp.ones((_T1, 1), _F32)
    sze_ref[0] = 1.0 + _dot(m1t_e, ones1, _HI)
    szo_ref[0] = 1.0 + _dot(m1t_o, ones1, _HI)


def _merge1(xe_ref, xo_ref, perm_ref, dstp_ref, x1w_ref):
    x1w_ref[0] = xo_ref[0, :, 0]

    def body(p, _):
        t = perm_ref[0, 0, p]
        dd = dstp_ref[0, 0, p]
        row = xe_ref[0, pl.ds(t, 1), 0]
        x1w_ref[0, pl.ds(dd, 1)] = x1w_ref[0, pl.ds(dd, 1)] + row
        return 0

    lax.fori_loop(0, _T1, body, 0)


def _round2(x1we4_ref, x1wo4_ref, x1we_ref, x1wo_ref, sze_ref, szo_ref,
            unm_ref, dst_ref):
    sze = sze_ref[0]
    szo = szo_ref[0]
    x1e = x1we_ref[0, :, 0].reshape(_T2, _C) / sze
    x1o = x1wo_ref[0, :, 0].reshape(_T2, _C) / szo
    mn_e = _metric(x1we4_ref[0, :, 0] / sze[:, :, None])
    mn_o = _metric(x1wo4_ref[0, :, 0] / szo[:, :, None])
    scores = _dot_t(mn_e, mn_o)
    scores_t2 = _dot_t(mn_o, mn_e)
    nmax_col = jnp.max(scores, axis=1, keepdims=True)
    nmax_row = _row_of_col(nmax_col)
    nidx_row2 = _rowmin_idx0(scores_t2)
    gt = (nmax_col > nmax_row).astype(jnp.int32)
    iota_j = lax.broadcasted_iota(jnp.int32, (_T2, _T2), 0)
    iota_t = lax.broadcasted_iota(jnp.int32, (_T2, _T2), 1)
    eqlt = ((nmax_col == nmax_row) & (iota_j < iota_t)).astype(jnp.int32)
    rank_row = jnp.sum(gt + eqlt, axis=0, keepdims=True)
    merged_row = rank_row < _R2

    x2e = x1e * sze
    x2o = x1o * szo

    iota_s2 = lax.broadcasted_iota(jnp.int32, (_T2, _T2), 0)
    m2t = (merged_row & (nidx_row2 == iota_s2)).astype(_F32)
    num_dst = x2o + _dot(m2t, x2e, _HI)
    den_dst = szo + _dot(m2t, sze, _HI)
    dst_ref[0] = num_dst / den_dst

    iota_u = lax.broadcasted_iota(jnp.int32, (_U2, _T2), 0)
    ut = ((~merged_row) & (rank_row - _R2 == iota_u)).astype(_F32)
    num_unm = _dot(ut, x2e, _HI)
    den_unm = _dot(ut, sze, _HI)
    unm_ref[0] = num_unm / den_unm


def kernel(x, target_num_token):
    del target_num_token
    x4 = x.reshape(_B, _P, _H, _D)

    x5 = x.reshape(_B, _T1, 2, _H, _D)

    spec = lambda s1, s2: pl.BlockSpec((1, s1, s2), lambda i: (i, 0, 0))
    perm, dstp, sze, szo = pl.pallas_call(
        _decide1,
        grid=(_B,),
        in_specs=[pl.BlockSpec((1, _T1, 1, _H, _D), lambda i: (i, 0, 0, 0, 0)),
                  pl.BlockSpec((1, _T1, 1, _H, _D), lambda i: (i, 0, 1, 0, 0))],
        out_specs=[spec(1, _T1), spec(1, _T1), spec(_T2, 1), spec(_T2, 1)],
        out_shape=[jax.ShapeDtypeStruct((_B, 1, _T1), jnp.int32),
                   jax.ShapeDtypeStruct((_B, 1, _T1), jnp.int32),
                   jax.ShapeDtypeStruct((_B, _T2, 1), _F32),
                   jax.ShapeDtypeStruct((_B, _T2, 1), _F32)],
    )(x5, x5)

    x6 = x.reshape(_B, _T1, 2, 8, 128)
    smem_spec = pl.BlockSpec((1, 1, _T1), lambda i: (i, 0, 0),
                             memory_space=pltpu.SMEM)
    x1w5 = pl.pallas_call(
        _merge1,
        grid=(_B,),
        in_specs=[pl.BlockSpec((1, _T1, 1, 8, 128), lambda i: (i, 0, 0, 0, 0)),
                  pl.BlockSpec((1, _T1, 1, 8, 128), lambda i: (i, 0, 1, 0, 0)),
                  smem_spec, smem_spec],
        out_specs=pl.BlockSpec((1, _T1, 8, 128), lambda i: (i, 0, 0, 0)),
        out_shape=jax.ShapeDtypeStruct((_B, _T1, 8, 128), _F32),
    )(x6, x6, perm, dstp)
    x1w = x1w5.reshape(_B, _T1, _C)

    x1w4 = x1w.reshape(_B, _T2, 2, _H, _D)
    x1w2 = x1w.reshape(_B, _T2, 2, 8, 128)
    spec4 = lambda p: pl.BlockSpec((1, _T2, 1, _H, _D),
                                   lambda i: (i, 0, p, 0, 0))
    spec2 = lambda p: pl.BlockSpec((1, _T2, 1, 8, 128),
                                   lambda i: (i, 0, p, 0, 0))
    out_unm, out_dst = pl.pallas_call(
        _round2,
        grid=(_B,),
        in_specs=[spec4(0), spec4(1), spec2(0), spec2(1),
                  spec(_T2, 1), spec(_T2, 1)],
        out_specs=[spec(_U2, _C), spec(_T2, _C)],
        out_shape=[jax.ShapeDtypeStruct((_B, _U2, _C), _F32),
                   jax.ShapeDtypeStruct((_B, _T2, _C), _F32)],
    )(x1w4, x1w4, x1w2, x1w2, sze, szo)

    return jnp.concatenate([out_unm, out_dst], axis=1)
